# baseline (speedup 1.0000x reference)
import jax
import jax.numpy as jnp
from jax.experimental import pallas as pl
from jax.experimental.pallas import tpu as pltpu

_ROWS = 1000000
_COLS = 64
_WROWS = _ROWS // 2
_WCOLS = 128
_BLOCK_ROWS = 4000


def _relu_scatter_body(x_ref, y_ref, c_ref, o_ref):
    o_ref[...] = jnp.maximum(c_ref[...], 0.0)

    @pl.when(pl.program_id(0) == 0)
    def _():
        m = x_ref[...] * y_ref[...]
        o_ref[0:1, 0:_COLS] = jnp.maximum(m, 0.0)


def kernel(x, y, cache):
    wide = jnp.reshape(cache, (_WROWS, _WCOLS))
    grid = _WROWS // _BLOCK_ROWS
    out = pl.pallas_call(
        _relu_scatter_body,
        grid=(grid,),
        in_specs=[
            pl.BlockSpec((1, _COLS), lambda i: (0, 0)),
            pl.BlockSpec((1, _COLS), lambda i: (0, 0)),
            pl.BlockSpec((_BLOCK_ROWS, _WCOLS), lambda i: (i, 0)),
        ],
        out_specs=pl.BlockSpec((_BLOCK_ROWS, _WCOLS), lambda i: (i, 0)),
        out_shape=jax.ShapeDtypeStruct((_WROWS, _WCOLS), jnp.float32),
        compiler_params=pltpu.CompilerParams(
            dimension_semantics=("arbitrary",),
        ),
    )(x, y, wide)
    return jnp.reshape(out, (_ROWS, _COLS))


# speedup vs baseline: 1.3785x; 1.3785x over previous
import jax
import jax.numpy as jnp
from jax.experimental import pallas as pl
from jax.experimental.pallas import tpu as pltpu

_ROWS = 1000000
_COLS = 64
_BLOCK_ROWS = 20000


def _relu_scatter_body(x_ref, y_ref, c_ref, o_ref):
    o_ref[...] = jnp.maximum(c_ref[...], 0.0)

    @pl.when(pl.program_id(0) == 0)
    def _():
        m = x_ref[...] * y_ref[...]
        o_ref[0:1, :] = jnp.maximum(m, 0.0)


def kernel(x, y, cache):
    grid = _ROWS // _BLOCK_ROWS
    return pl.pallas_call(
        _relu_scatter_body,
        grid=(grid,),
        in_specs=[
            pl.BlockSpec((1, _COLS), lambda i: (0, 0)),
            pl.BlockSpec((1, _COLS), lambda i: (0, 0)),
            pl.BlockSpec((_BLOCK_ROWS, _COLS), lambda i: (i, 0)),
        ],
        out_specs=pl.BlockSpec((_BLOCK_ROWS, _COLS), lambda i: (i, 0)),
        out_shape=jax.ShapeDtypeStruct((_ROWS, _COLS), jnp.float32),
        compiler_params=pltpu.CompilerParams(
            dimension_semantics=("arbitrary",),
        ),
    )(x, y, cache)


# speedup vs baseline: 1.3796x; 1.0008x over previous
import jax
import jax.numpy as jnp
from jax.experimental import pallas as pl
from jax.experimental.pallas import tpu as pltpu

_ROWS = 1000000
_COLS = 64
_BLOCK_ROWS = 20000


def _relu_scatter_body(x_ref, y_ref, c_ref, o_ref):
    o_ref[...] = jnp.maximum(c_ref[...], 0.0)

    @pl.when(pl.program_id(0) == 0)
    def _():
        m = x_ref[...] * y_ref[...]
        o_ref[0:1, :] = jnp.maximum(m, 0.0)


def kernel(x, y, cache):
    grid = _ROWS // _BLOCK_ROWS
    return pl.pallas_call(
        _relu_scatter_body,
        grid=(grid,),
        in_specs=[
            pl.BlockSpec((1, _COLS), lambda i: (0, 0)),
            pl.BlockSpec((1, _COLS), lambda i: (0, 0)),
            pl.BlockSpec((_BLOCK_ROWS, _COLS), lambda i: (i, 0)),
        ],
        out_specs=pl.BlockSpec((_BLOCK_ROWS, _COLS), lambda i: (i, 0)),
        out_shape=jax.ShapeDtypeStruct((_ROWS, _COLS), jnp.float32),
        compiler_params=pltpu.CompilerParams(
            dimension_semantics=("parallel",),
        ),
    )(x, y, cache)
